# Initial kernel scaffold; baseline (speedup 1.0000x reference)
import jax
import jax.numpy as jnp
from jax.experimental import pallas as pl


def kernel(lst, gui, mask, fc1_w, fc1_b, fc2_w, fc2_b, se1_w, se1_b, se2_w, se2_b, conv1_w, conv1_b, conv2_w, conv2_b):
    raise NotImplementedError("write your pallas kernel here")


# speedup vs baseline: 1.1482x; 1.1482x over previous
import functools

import jax
import jax.numpy as jnp
from jax.experimental import pallas as pl


def _pool_body(lst_ref, gui_ref, out_ref):
    i = pl.program_id(0)

    @pl.when(i == 0)
    def _():
        out_ref[...] = jnp.zeros_like(out_ref)

    s_l = jnp.sum(lst_ref[...], axis=1, keepdims=True)
    s_g = jnp.sum(gui_ref[...], axis=1, keepdims=True)
    out_ref[...] += jnp.concatenate([s_l, s_g], axis=1)


def _select_body(sums_ref, mask_ref, fc1_w_ref, fc1_b_ref, fc2_w_ref,
                 fc2_b_ref, se1_w_ref, se1_b_ref, se2_w_ref, se2_b_ref,
                 conv1_w_ref, conv1_b_ref, conv2_w_ref, conv2_b_ref,
                 b_lst_ref, b_gui_ref, vec_ref, *, hw, k):
    c = fc1_w_ref.shape[0]
    hid = jax.nn.relu(
        jnp.dot(fc1_w_ref[...], mask_ref[...],
                preferred_element_type=jnp.float32) + fc1_b_ref[...])
    mask1 = jax.nn.sigmoid(
        jnp.dot(fc2_w_ref[...], hid,
                preferred_element_type=jnp.float32) + fc2_b_ref[...])
    pooled = jnp.concatenate(
        [sums_ref[:, 0:1], sums_ref[:, 1:2]], axis=0) * (1.0 / hw)
    se_h = jax.nn.relu(
        jnp.dot(se1_w_ref[...], pooled,
                preferred_element_type=jnp.float32) + se1_b_ref[...])
    mask2 = jax.nn.sigmoid(
        jnp.dot(se2_w_ref[...], se_h,
                preferred_element_type=jnp.float32) + se2_b_ref[...])
    m = mask1 * mask2

    mm = jnp.broadcast_to(m, (c, c))
    mt = mm.T
    ii = jax.lax.broadcasted_iota(jnp.int32, (c, c), 0)
    jj = jax.lax.broadcasted_iota(jnp.int32, (c, c), 1)
    beats = (mt > mm) | ((mt == mm) & (jj < ii))
    rank = jnp.sum(beats.astype(jnp.float32), axis=1, keepdims=True)
    sel = (rank < k).astype(jnp.float32)

    lower = (jj < ii).astype(jnp.float32)
    pos = jnp.dot(lower, sel, preferred_element_type=jnp.float32,
                  precision=jax.lax.Precision.HIGHEST)
    oo = jax.lax.broadcasted_iota(jnp.int32, (c, k), 1).astype(jnp.float32)
    scat = sel * (pos == oo).astype(jnp.float32)

    def mix(conv_w, conv_b):
        hi = jax.lax.Precision.HIGHEST
        t = jnp.dot(scat, conv_w, preferred_element_type=jnp.float32,
                    precision=hi)
        b = jax.lax.dot_general(t, scat, (((1,), (1,)), ((), ())),
                                preferred_element_type=jnp.float32,
                                precision=hi)
        bias = jnp.dot(scat, conv_b, preferred_element_type=jnp.float32,
                       precision=hi)
        return b, bias

    b_lst, bias_lst = mix(conv2_w_ref[...], conv2_b_ref[...])
    b_gui, bias_gui = mix(conv1_w_ref[...], conv1_b_ref[...])
    b_lst_ref[...] = b_lst
    b_gui_ref[...] = b_gui
    vec_ref[...] = jnp.concatenate(
        [m, 1.0 - sel, bias_lst, bias_gui], axis=1)


def _transform_body(lst_ref, gui_ref, b_lst_ref, b_gui_ref, vec_ref,
                    out_lst_ref, out_gui_ref):
    notsel = vec_ref[:, 1:2]
    bias_lst = vec_ref[:, 2:3]
    bias_gui = vec_ref[:, 3:4]
    lst_t = lst_ref[...]
    gui_t = gui_ref[...]
    out_lst_ref[...] = (notsel * lst_t + bias_lst +
                        jnp.dot(b_lst_ref[...], gui_t,
                                preferred_element_type=jnp.float32))
    out_gui_ref[...] = (notsel * gui_t + bias_gui +
                        jnp.dot(b_gui_ref[...], lst_t,
                                preferred_element_type=jnp.float32))


def kernel(lst, gui, mask, fc1_w, fc1_b, fc2_w, fc2_b, se1_w, se1_b,
           se2_w, se2_b, conv1_w, conv1_b, conv2_w, conv2_b):
    n, c, h, w = lst.shape
    hw = h * w
    k = c // 2
    lst2 = lst.reshape(c, hw)
    gui2 = gui.reshape(c, hw)

    tile = min(4096, hw)
    grid = hw // tile

    sums = pl.pallas_call(
        _pool_body,
        grid=(grid,),
        in_specs=[
            pl.BlockSpec((c, tile), lambda i: (0, i)),
            pl.BlockSpec((c, tile), lambda i: (0, i)),
        ],
        out_specs=pl.BlockSpec((c, 2), lambda i: (0, 0)),
        out_shape=jax.ShapeDtypeStruct((c, 2), jnp.float32),
    )(lst2, gui2)

    cr = se1_w.shape[0]
    full = lambda s: pl.BlockSpec(s, lambda: (0,) * len(s))
    b_lst, b_gui, vec = pl.pallas_call(
        functools.partial(_select_body, hw=float(hw), k=k),
        grid=(),
        in_specs=[full((c, 2)), full((mask.shape[1], 1)),
                  full((c, mask.shape[1])), full((c, 1)),
                  full((c, c)), full((c, 1)),
                  full((cr, 2 * c)), full((cr, 1)),
                  full((c, cr)), full((c, 1)),
                  full((k, k)), full((k, 1)),
                  full((k, k)), full((k, 1))],
        out_specs=[full((c, c)), full((c, c)), full((c, 4))],
        out_shape=[jax.ShapeDtypeStruct((c, c), jnp.float32),
                   jax.ShapeDtypeStruct((c, c), jnp.float32),
                   jax.ShapeDtypeStruct((c, 4), jnp.float32)],
    )(sums, mask.reshape(mask.shape[1], 1),
      fc1_w, fc1_b.reshape(c, 1), fc2_w, fc2_b.reshape(c, 1),
      se1_w, se1_b.reshape(cr, 1), se2_w, se2_b.reshape(c, 1),
      conv1_w, conv1_b.reshape(k, 1), conv2_w, conv2_b.reshape(k, 1))

    out_lst2, out_gui2 = pl.pallas_call(
        _transform_body,
        grid=(grid,),
        in_specs=[
            pl.BlockSpec((c, tile), lambda i: (0, i)),
            pl.BlockSpec((c, tile), lambda i: (0, i)),
            pl.BlockSpec((c, c), lambda i: (0, 0)),
            pl.BlockSpec((c, c), lambda i: (0, 0)),
            pl.BlockSpec((c, 4), lambda i: (0, 0)),
        ],
        out_specs=[
            pl.BlockSpec((c, tile), lambda i: (0, i)),
            pl.BlockSpec((c, tile), lambda i: (0, i)),
        ],
        out_shape=[jax.ShapeDtypeStruct((c, hw), jnp.float32),
                   jax.ShapeDtypeStruct((c, hw), jnp.float32)],
    )(lst2, gui2, b_lst, b_gui, vec)

    m_out = vec[:, 0].reshape(n, c)
    return (out_lst2.reshape(n, c, h, w), out_gui2.reshape(n, c, h, w),
            m_out)


# speedup vs baseline: 2.2693x; 1.9764x over previous
import functools

import jax
import jax.numpy as jnp
from jax.experimental import pallas as pl


def _pool_body(lst_ref, gui_ref, out_ref):
    i = pl.program_id(0)

    @pl.when(i == 0)
    def _():
        out_ref[...] = jnp.zeros_like(out_ref)

    s_l = jnp.sum(jnp.sum(lst_ref[0], axis=2), axis=1, keepdims=True)
    s_g = jnp.sum(jnp.sum(gui_ref[0], axis=2), axis=1, keepdims=True)
    out_ref[...] += jnp.concatenate([s_l, s_g], axis=1)


def _select_body(sums_ref, mask_ref, fc1_w_ref, fc1_b_ref, fc2_w_ref,
                 fc2_b_ref, se1_w_ref, se1_b_ref, se2_w_ref, se2_b_ref,
                 conv1_w_ref, conv1_b_ref, conv2_w_ref, conv2_b_ref,
                 b_lst_ref, b_gui_ref, vec_ref, *, hw, k):
    c = fc1_w_ref.shape[0]
    hid = jax.nn.relu(
        jnp.dot(fc1_w_ref[...], mask_ref[...],
                preferred_element_type=jnp.float32) + fc1_b_ref[...])
    mask1 = jax.nn.sigmoid(
        jnp.dot(fc2_w_ref[...], hid,
                preferred_element_type=jnp.float32) + fc2_b_ref[...])
    pooled = jnp.concatenate(
        [sums_ref[:, 0:1], sums_ref[:, 1:2]], axis=0) * (1.0 / hw)
    se_h = jax.nn.relu(
        jnp.dot(se1_w_ref[...], pooled,
                preferred_element_type=jnp.float32) + se1_b_ref[...])
    mask2 = jax.nn.sigmoid(
        jnp.dot(se2_w_ref[...], se_h,
                preferred_element_type=jnp.float32) + se2_b_ref[...])
    m = mask1 * mask2

    mm = jnp.broadcast_to(m, (c, c))
    mt = mm.T
    ii = jax.lax.broadcasted_iota(jnp.int32, (c, c), 0)
    jj = jax.lax.broadcasted_iota(jnp.int32, (c, c), 1)
    beats = (mt > mm) | ((mt == mm) & (jj < ii))
    rank = jnp.sum(beats.astype(jnp.float32), axis=1, keepdims=True)
    sel = (rank < k).astype(jnp.float32)

    lower = (jj < ii).astype(jnp.float32)
    pos = jnp.dot(lower, sel, preferred_element_type=jnp.float32,
                  precision=jax.lax.Precision.HIGHEST)
    oo = jax.lax.broadcasted_iota(jnp.int32, (c, k), 1).astype(jnp.float32)
    scat = sel * (pos == oo).astype(jnp.float32)

    def mix(conv_w, conv_b):
        hi = jax.lax.Precision.HIGHEST
        t = jnp.dot(scat, conv_w, preferred_element_type=jnp.float32,
                    precision=hi)
        b = jax.lax.dot_general(t, scat, (((1,), (1,)), ((), ())),
                                preferred_element_type=jnp.float32,
                                precision=hi)
        bias = jnp.dot(scat, conv_b, preferred_element_type=jnp.float32,
                       precision=hi)
        return b, bias

    b_lst, bias_lst = mix(conv2_w_ref[...], conv2_b_ref[...])
    b_gui, bias_gui = mix(conv1_w_ref[...], conv1_b_ref[...])
    b_lst_ref[...] = b_lst
    b_gui_ref[...] = b_gui
    vec_ref[...] = jnp.concatenate(
        [m, 1.0 - sel, bias_lst, bias_gui], axis=1)


def _transform_body(lst_ref, gui_ref, b_lst_ref, b_gui_ref, vec_ref,
                    out_lst_ref, out_gui_ref, *, hb):
    notsel = vec_ref[:, 1:2]
    bias_lst = vec_ref[:, 2:3]
    bias_gui = vec_ref[:, 3:4]
    b_lst = b_lst_ref[...]
    b_gui = b_gui_ref[...]
    for r in range(hb):
        lst_t = lst_ref[0, :, r, :]
        gui_t = gui_ref[0, :, r, :]
        out_lst_ref[0, :, r, :] = (notsel * lst_t + bias_lst +
                                   jnp.dot(b_lst, gui_t,
                                           preferred_element_type=jnp.float32))
        out_gui_ref[0, :, r, :] = (notsel * gui_t + bias_gui +
                                   jnp.dot(b_gui, lst_t,
                                           preferred_element_type=jnp.float32))


def kernel(lst, gui, mask, fc1_w, fc1_b, fc2_w, fc2_b, se1_w, se1_b,
           se2_w, se2_b, conv1_w, conv1_b, conv2_w, conv2_b):
    n, c, h, w = lst.shape
    hw = h * w
    k = c // 2

    hb = min(8, h)
    grid = h // hb
    big_spec = pl.BlockSpec((1, c, hb, w), lambda i: (0, 0, i, 0))

    sums = pl.pallas_call(
        _pool_body,
        grid=(grid,),
        in_specs=[big_spec, big_spec],
        out_specs=pl.BlockSpec((c, 2), lambda i: (0, 0)),
        out_shape=jax.ShapeDtypeStruct((c, 2), jnp.float32),
    )(lst, gui)

    cr = se1_w.shape[0]
    full = lambda s: pl.BlockSpec(s, lambda: (0,) * len(s))
    b_lst, b_gui, vec = pl.pallas_call(
        functools.partial(_select_body, hw=float(hw), k=k),
        grid=(),
        in_specs=[full((c, 2)), full((mask.shape[1], 1)),
                  full((c, mask.shape[1])), full((c, 1)),
                  full((c, c)), full((c, 1)),
                  full((cr, 2 * c)), full((cr, 1)),
                  full((c, cr)), full((c, 1)),
                  full((k, k)), full((k, 1)),
                  full((k, k)), full((k, 1))],
        out_specs=[full((c, c)), full((c, c)), full((c, 4))],
        out_shape=[jax.ShapeDtypeStruct((c, c), jnp.float32),
                   jax.ShapeDtypeStruct((c, c), jnp.float32),
                   jax.ShapeDtypeStruct((c, 4), jnp.float32)],
    )(sums, mask.reshape(mask.shape[1], 1),
      fc1_w, fc1_b.reshape(c, 1), fc2_w, fc2_b.reshape(c, 1),
      se1_w, se1_b.reshape(cr, 1), se2_w, se2_b.reshape(c, 1),
      conv1_w, conv1_b.reshape(k, 1), conv2_w, conv2_b.reshape(k, 1))

    out_lst, out_gui = pl.pallas_call(
        functools.partial(_transform_body, hb=hb),
        grid=(grid,),
        in_specs=[
            big_spec,
            big_spec,
            pl.BlockSpec((c, c), lambda i: (0, 0)),
            pl.BlockSpec((c, c), lambda i: (0, 0)),
            pl.BlockSpec((c, 4), lambda i: (0, 0)),
        ],
        out_specs=[big_spec, big_spec],
        out_shape=[jax.ShapeDtypeStruct((n, c, h, w), jnp.float32),
                   jax.ShapeDtypeStruct((n, c, h, w), jnp.float32)],
    )(lst, gui, b_lst, b_gui, vec)

    m_out = vec[:, 0].reshape(n, c)
    return out_lst, out_gui, m_out


# speedup vs baseline: 2.5977x; 1.1447x over previous
import functools

import jax
import jax.numpy as jnp
from jax.experimental import pallas as pl


def _pool_body(lst_ref, gui_ref, out_ref):
    i = pl.program_id(0)

    @pl.when(i == 0)
    def _():
        out_ref[...] = jnp.zeros_like(out_ref)

    s_l = jnp.sum(jnp.sum(lst_ref[0], axis=2), axis=1, keepdims=True)
    s_g = jnp.sum(jnp.sum(gui_ref[0], axis=2), axis=1, keepdims=True)
    out_ref[...] += jnp.concatenate([s_l, s_g], axis=1)


def _select_body(sums_ref, mask_ref, fc1_w_ref, fc1_b_ref, fc2_w_ref,
                 fc2_b_ref, se1_w_ref, se1_b_ref, se2_w_ref, se2_b_ref,
                 conv1_w_ref, conv1_b_ref, conv2_w_ref, conv2_b_ref,
                 b_lst_ref, b_gui_ref, vec_ref, *, hw, k):
    c = fc1_w_ref.shape[0]
    hid = jax.nn.relu(
        jnp.dot(fc1_w_ref[...], mask_ref[...],
                preferred_element_type=jnp.float32) + fc1_b_ref[...])
    mask1 = jax.nn.sigmoid(
        jnp.dot(fc2_w_ref[...], hid,
                preferred_element_type=jnp.float32) + fc2_b_ref[...])
    pooled = jnp.concatenate(
        [sums_ref[:, 0:1], sums_ref[:, 1:2]], axis=0) * (1.0 / hw)
    se_h = jax.nn.relu(
        jnp.dot(se1_w_ref[...], pooled,
                preferred_element_type=jnp.float32) + se1_b_ref[...])
    mask2 = jax.nn.sigmoid(
        jnp.dot(se2_w_ref[...], se_h,
                preferred_element_type=jnp.float32) + se2_b_ref[...])
    m = mask1 * mask2

    mm = jnp.broadcast_to(m, (c, c))
    mt = mm.T
    ii = jax.lax.broadcasted_iota(jnp.int32, (c, c), 0)
    jj = jax.lax.broadcasted_iota(jnp.int32, (c, c), 1)
    beats = (mt > mm) | ((mt == mm) & (jj < ii))
    rank = jnp.sum(beats.astype(jnp.float32), axis=1, keepdims=True)
    sel = (rank < k).astype(jnp.float32)

    lower = (jj < ii).astype(jnp.float32)
    pos = jnp.dot(lower, sel, preferred_element_type=jnp.float32,
                  precision=jax.lax.Precision.HIGHEST)
    oo = jax.lax.broadcasted_iota(jnp.int32, (c, k), 1).astype(jnp.float32)
    scat = sel * (pos == oo).astype(jnp.float32)

    def mix(conv_w, conv_b):
        hi = jax.lax.Precision.HIGHEST
        t = jnp.dot(scat, conv_w, preferred_element_type=jnp.float32,
                    precision=hi)
        b = jax.lax.dot_general(t, scat, (((1,), (1,)), ((), ())),
                                preferred_element_type=jnp.float32,
                                precision=hi)
        bias = jnp.dot(scat, conv_b, preferred_element_type=jnp.float32,
                       precision=hi)
        return b, bias

    b_lst, bias_lst = mix(conv2_w_ref[...], conv2_b_ref[...])
    b_gui, bias_gui = mix(conv1_w_ref[...], conv1_b_ref[...])
    b_lst_ref[...] = b_lst
    b_gui_ref[...] = b_gui
    vec_ref[...] = jnp.concatenate(
        [m, 1.0 - sel, bias_lst, bias_gui], axis=1)


def _transform_body(lst_ref, gui_ref, b_lst_ref, b_gui_ref, vec_ref,
                    out_lst_ref, out_gui_ref, *, hb):
    notsel = vec_ref[:, 1:2]
    bias_lst = vec_ref[:, 2:3]
    bias_gui = vec_ref[:, 3:4]
    b_lst = b_lst_ref[...]
    b_gui = b_gui_ref[...]
    for r in range(hb):
        lst_t = lst_ref[0, :, r, :]
        gui_t = gui_ref[0, :, r, :]
        out_lst_ref[0, :, r, :] = (notsel * lst_t + bias_lst +
                                   jnp.dot(b_lst, gui_t,
                                           preferred_element_type=jnp.float32))
        out_gui_ref[0, :, r, :] = (notsel * gui_t + bias_gui +
                                   jnp.dot(b_gui, lst_t,
                                           preferred_element_type=jnp.float32))


def kernel(lst, gui, mask, fc1_w, fc1_b, fc2_w, fc2_b, se1_w, se1_b,
           se2_w, se2_b, conv1_w, conv1_b, conv2_w, conv2_b):
    n, c, h, w = lst.shape
    hw = h * w
    k = c // 2

    hb = min(16, h)
    grid = h // hb
    big_spec = pl.BlockSpec((1, c, hb, w), lambda i: (0, 0, i, 0))

    sums = pl.pallas_call(
        _pool_body,
        grid=(grid,),
        in_specs=[big_spec, big_spec],
        out_specs=pl.BlockSpec((c, 2), lambda i: (0, 0)),
        out_shape=jax.ShapeDtypeStruct((c, 2), jnp.float32),
    )(lst, gui)

    cr = se1_w.shape[0]
    full = lambda s: pl.BlockSpec(s, lambda: (0,) * len(s))
    b_lst, b_gui, vec = pl.pallas_call(
        functools.partial(_select_body, hw=float(hw), k=k),
        grid=(),
        in_specs=[full((c, 2)), full((mask.shape[1], 1)),
                  full((c, mask.shape[1])), full((c, 1)),
                  full((c, c)), full((c, 1)),
                  full((cr, 2 * c)), full((cr, 1)),
                  full((c, cr)), full((c, 1)),
                  full((k, k)), full((k, 1)),
                  full((k, k)), full((k, 1))],
        out_specs=[full((c, c)), full((c, c)), full((c, 4))],
        out_shape=[jax.ShapeDtypeStruct((c, c), jnp.float32),
                   jax.ShapeDtypeStruct((c, c), jnp.float32),
                   jax.ShapeDtypeStruct((c, 4), jnp.float32)],
    )(sums, mask.reshape(mask.shape[1], 1),
      fc1_w, fc1_b.reshape(c, 1), fc2_w, fc2_b.reshape(c, 1),
      se1_w, se1_b.reshape(cr, 1), se2_w, se2_b.reshape(c, 1),
      conv1_w, conv1_b.reshape(k, 1), conv2_w, conv2_b.reshape(k, 1))

    out_lst, out_gui = pl.pallas_call(
        functools.partial(_transform_body, hb=hb),
        grid=(grid,),
        in_specs=[
            big_spec,
            big_spec,
            pl.BlockSpec((c, c), lambda i: (0, 0)),
            pl.BlockSpec((c, c), lambda i: (0, 0)),
            pl.BlockSpec((c, 4), lambda i: (0, 0)),
        ],
        out_specs=[big_spec, big_spec],
        out_shape=[jax.ShapeDtypeStruct((n, c, h, w), jnp.float32),
                   jax.ShapeDtypeStruct((n, c, h, w), jnp.float32)],
    )(lst, gui, b_lst, b_gui, vec)

    m_out = vec[:, 0].reshape(n, c)
    return out_lst, out_gui, m_out


# speedup vs baseline: 2.6797x; 1.0316x over previous
import functools

import jax
import jax.numpy as jnp
from jax.experimental import pallas as pl


def _pool_body(lst_ref, gui_ref, out_ref):
    i = pl.program_id(0)

    @pl.when(i == 0)
    def _():
        out_ref[...] = jnp.zeros_like(out_ref)

    s_l = jnp.sum(jnp.sum(lst_ref[0], axis=2), axis=1, keepdims=True)
    s_g = jnp.sum(jnp.sum(gui_ref[0], axis=2), axis=1, keepdims=True)
    out_ref[...] += jnp.concatenate([s_l, s_g], axis=1)


def _select_body(sums_ref, mask_ref, fc1_w_ref, fc1_b_ref, fc2_w_ref,
                 fc2_b_ref, se1_w_ref, se1_b_ref, se2_w_ref, se2_b_ref,
                 conv1_w_ref, conv1_b_ref, conv2_w_ref, conv2_b_ref,
                 b_lst_ref, b_gui_ref, vec_ref, *, hw, k):
    c = fc1_w_ref.shape[0]
    hid = jax.nn.relu(
        jnp.dot(fc1_w_ref[...], mask_ref[...],
                preferred_element_type=jnp.float32) + fc1_b_ref[...])
    mask1 = jax.nn.sigmoid(
        jnp.dot(fc2_w_ref[...], hid,
                preferred_element_type=jnp.float32) + fc2_b_ref[...])
    pooled = jnp.concatenate(
        [sums_ref[:, 0:1], sums_ref[:, 1:2]], axis=0) * (1.0 / hw)
    se_h = jax.nn.relu(
        jnp.dot(se1_w_ref[...], pooled,
                preferred_element_type=jnp.float32) + se1_b_ref[...])
    mask2 = jax.nn.sigmoid(
        jnp.dot(se2_w_ref[...], se_h,
                preferred_element_type=jnp.float32) + se2_b_ref[...])
    m = mask1 * mask2

    mm = jnp.broadcast_to(m, (c, c))
    mt = mm.T
    ii = jax.lax.broadcasted_iota(jnp.int32, (c, c), 0)
    jj = jax.lax.broadcasted_iota(jnp.int32, (c, c), 1)
    beats = (mt > mm) | ((mt == mm) & (jj < ii))
    rank = jnp.sum(beats.astype(jnp.float32), axis=1, keepdims=True)
    sel = (rank < k).astype(jnp.float32)

    lower = (jj < ii).astype(jnp.float32)
    pos = jnp.dot(lower, sel, preferred_element_type=jnp.float32,
                  precision=jax.lax.Precision.HIGHEST)
    oo = jax.lax.broadcasted_iota(jnp.int32, (c, k), 1).astype(jnp.float32)
    scat = sel * (pos == oo).astype(jnp.float32)

    def mix(conv_w, conv_b):
        hi = jax.lax.Precision.HIGHEST
        t = jnp.dot(scat, conv_w, preferred_element_type=jnp.float32,
                    precision=hi)
        b = jax.lax.dot_general(t, scat, (((1,), (1,)), ((), ())),
                                preferred_element_type=jnp.float32,
                                precision=hi)
        bias = jnp.dot(scat, conv_b, preferred_element_type=jnp.float32,
                       precision=hi)
        return b, bias

    b_lst, bias_lst = mix(conv2_w_ref[...], conv2_b_ref[...])
    b_gui, bias_gui = mix(conv1_w_ref[...], conv1_b_ref[...])
    b_lst_ref[...] = b_lst
    b_gui_ref[...] = b_gui
    vec_ref[...] = jnp.concatenate(
        [m, 1.0 - sel, bias_lst, bias_gui], axis=1)


def _transform_body(lst_ref, gui_ref, b_lst_ref, b_gui_ref, vec_ref,
                    out_lst_ref, out_gui_ref, *, hb):
    notsel = vec_ref[:, 1:2]
    bias_lst = vec_ref[:, 2:3]
    bias_gui = vec_ref[:, 3:4]
    b_lst = b_lst_ref[...]
    b_gui = b_gui_ref[...]
    for r in range(hb):
        lst_t = lst_ref[0, :, r, :]
        gui_t = gui_ref[0, :, r, :]
        out_lst_ref[0, :, r, :] = (notsel * lst_t + bias_lst +
                                   jnp.dot(b_lst, gui_t,
                                           preferred_element_type=jnp.float32))
        out_gui_ref[0, :, r, :] = (notsel * gui_t + bias_gui +
                                   jnp.dot(b_gui, lst_t,
                                           preferred_element_type=jnp.float32))


def kernel(lst, gui, mask, fc1_w, fc1_b, fc2_w, fc2_b, se1_w, se1_b,
           se2_w, se2_b, conv1_w, conv1_b, conv2_w, conv2_b):
    n, c, h, w = lst.shape
    hw = h * w
    k = c // 2

    hb = min(32, h)
    grid = h // hb
    big_spec = pl.BlockSpec((1, c, hb, w), lambda i: (0, 0, i, 0))

    sums = pl.pallas_call(
        _pool_body,
        grid=(grid,),
        in_specs=[big_spec, big_spec],
        out_specs=pl.BlockSpec((c, 2), lambda i: (0, 0)),
        out_shape=jax.ShapeDtypeStruct((c, 2), jnp.float32),
    )(lst, gui)

    cr = se1_w.shape[0]
    full = lambda s: pl.BlockSpec(s, lambda: (0,) * len(s))
    b_lst, b_gui, vec = pl.pallas_call(
        functools.partial(_select_body, hw=float(hw), k=k),
        grid=(),
        in_specs=[full((c, 2)), full((mask.shape[1], 1)),
                  full((c, mask.shape[1])), full((c, 1)),
                  full((c, c)), full((c, 1)),
                  full((cr, 2 * c)), full((cr, 1)),
                  full((c, cr)), full((c, 1)),
                  full((k, k)), full((k, 1)),
                  full((k, k)), full((k, 1))],
        out_specs=[full((c, c)), full((c, c)), full((c, 4))],
        out_shape=[jax.ShapeDtypeStruct((c, c), jnp.float32),
                   jax.ShapeDtypeStruct((c, c), jnp.float32),
                   jax.ShapeDtypeStruct((c, 4), jnp.float32)],
    )(sums, mask.reshape(mask.shape[1], 1),
      fc1_w, fc1_b.reshape(c, 1), fc2_w, fc2_b.reshape(c, 1),
      se1_w, se1_b.reshape(cr, 1), se2_w, se2_b.reshape(c, 1),
      conv1_w, conv1_b.reshape(k, 1), conv2_w, conv2_b.reshape(k, 1))

    out_lst, out_gui = pl.pallas_call(
        functools.partial(_transform_body, hb=hb),
        grid=(grid,),
        in_specs=[
            big_spec,
            big_spec,
            pl.BlockSpec((c, c), lambda i: (0, 0)),
            pl.BlockSpec((c, c), lambda i: (0, 0)),
            pl.BlockSpec((c, 4), lambda i: (0, 0)),
        ],
        out_specs=[big_spec, big_spec],
        out_shape=[jax.ShapeDtypeStruct((n, c, h, w), jnp.float32),
                   jax.ShapeDtypeStruct((n, c, h, w), jnp.float32)],
    )(lst, gui, b_lst, b_gui, vec)

    m_out = vec[:, 0].reshape(n, c)
    return out_lst, out_gui, m_out


# speedup vs baseline: 2.8043x; 1.0465x over previous
import functools

import jax
import jax.numpy as jnp
from jax.experimental import pallas as pl


def _pool_body(lst_ref, gui_ref, out_ref):
    i = pl.program_id(0)

    @pl.when(i == 0)
    def _():
        out_ref[...] = jnp.zeros_like(out_ref)

    s_l = jnp.sum(jnp.sum(lst_ref[0], axis=2), axis=1, keepdims=True)
    s_g = jnp.sum(jnp.sum(gui_ref[0], axis=2), axis=1, keepdims=True)
    out_ref[...] += jnp.concatenate([s_l, s_g], axis=1)


def _select_body(sums_ref, mask_ref, fc1_w_ref, fc1_b_ref, fc2_w_ref,
                 fc2_b_ref, se1_w_ref, se1_b_ref, se2_w_ref, se2_b_ref,
                 conv1_w_ref, conv1_b_ref, conv2_w_ref, conv2_b_ref,
                 a_ref, vec_ref, *, hw, k):
    c = fc1_w_ref.shape[0]
    hid = jax.nn.relu(
        jnp.dot(fc1_w_ref[...], mask_ref[...],
                preferred_element_type=jnp.float32) + fc1_b_ref[...])
    mask1 = jax.nn.sigmoid(
        jnp.dot(fc2_w_ref[...], hid,
                preferred_element_type=jnp.float32) + fc2_b_ref[...])
    pooled = jnp.concatenate(
        [sums_ref[:, 0:1], sums_ref[:, 1:2]], axis=0) * (1.0 / hw)
    se_h = jax.nn.relu(
        jnp.dot(se1_w_ref[...], pooled,
                preferred_element_type=jnp.float32) + se1_b_ref[...])
    mask2 = jax.nn.sigmoid(
        jnp.dot(se2_w_ref[...], se_h,
                preferred_element_type=jnp.float32) + se2_b_ref[...])
    m = mask1 * mask2

    mm = jnp.broadcast_to(m, (c, c))
    mt = mm.T
    ii = jax.lax.broadcasted_iota(jnp.int32, (c, c), 0)
    jj = jax.lax.broadcasted_iota(jnp.int32, (c, c), 1)
    beats = (mt > mm) | ((mt == mm) & (jj < ii))
    rank = jnp.sum(beats.astype(jnp.float32), axis=1, keepdims=True)
    sel = (rank < k).astype(jnp.float32)

    lower = (jj < ii).astype(jnp.float32)
    pos = jnp.dot(lower, sel, preferred_element_type=jnp.float32,
                  precision=jax.lax.Precision.HIGHEST)
    oo = jax.lax.broadcasted_iota(jnp.int32, (c, k), 1).astype(jnp.float32)
    scat = sel * (pos == oo).astype(jnp.float32)

    def mix(conv_w, conv_b):
        hi = jax.lax.Precision.HIGHEST
        t = jnp.dot(scat, conv_w, preferred_element_type=jnp.float32,
                    precision=hi)
        b = jax.lax.dot_general(t, scat, (((1,), (1,)), ((), ())),
                                preferred_element_type=jnp.float32,
                                precision=hi)
        bias = jnp.dot(scat, conv_b, preferred_element_type=jnp.float32,
                       precision=hi)
        return b, bias

    b_lst, bias_lst = mix(conv2_w_ref[...], conv2_b_ref[...])
    b_gui, bias_gui = mix(conv1_w_ref[...], conv1_b_ref[...])
    dn = (ii == jj).astype(jnp.float32) * (1.0 - sel)
    a_ref[...] = jnp.concatenate(
        [jnp.concatenate([dn, b_lst], axis=1),
         jnp.concatenate([b_gui, dn], axis=1)], axis=0)
    vec_ref[...] = jnp.concatenate(
        [jnp.concatenate([m, m], axis=0),
         jnp.concatenate([bias_lst, bias_gui], axis=0)], axis=1)


def _transform_body(lst_ref, gui_ref, a_ref, vec_ref,
                    out_lst_ref, out_gui_ref, *, hb):
    c = lst_ref.shape[1]
    bias = vec_ref[:, 1:2]
    a = a_ref[...]
    for r in range(hb):
        cat = jnp.concatenate([lst_ref[0, :, r, :], gui_ref[0, :, r, :]],
                              axis=0)
        out = jnp.dot(a, cat, preferred_element_type=jnp.float32) + bias
        out_lst_ref[0, :, r, :] = out[:c]
        out_gui_ref[0, :, r, :] = out[c:]


def kernel(lst, gui, mask, fc1_w, fc1_b, fc2_w, fc2_b, se1_w, se1_b,
           se2_w, se2_b, conv1_w, conv1_b, conv2_w, conv2_b):
    n, c, h, w = lst.shape
    hw = h * w
    k = c // 2

    hb = min(32, h)
    grid = h // hb
    big_spec = pl.BlockSpec((1, c, hb, w), lambda i: (0, 0, i, 0))

    sums = pl.pallas_call(
        _pool_body,
        grid=(grid,),
        in_specs=[big_spec, big_spec],
        out_specs=pl.BlockSpec((c, 2), lambda i: (0, 0)),
        out_shape=jax.ShapeDtypeStruct((c, 2), jnp.float32),
    )(lst, gui)

    cr = se1_w.shape[0]
    full = lambda s: pl.BlockSpec(s, lambda: (0,) * len(s))
    a_mix, vec = pl.pallas_call(
        functools.partial(_select_body, hw=float(hw), k=k),
        grid=(),
        in_specs=[full((c, 2)), full((mask.shape[1], 1)),
                  full((c, mask.shape[1])), full((c, 1)),
                  full((c, c)), full((c, 1)),
                  full((cr, 2 * c)), full((cr, 1)),
                  full((c, cr)), full((c, 1)),
                  full((k, k)), full((k, 1)),
                  full((k, k)), full((k, 1))],
        out_specs=[full((2 * c, 2 * c)), full((2 * c, 2))],
        out_shape=[jax.ShapeDtypeStruct((2 * c, 2 * c), jnp.float32),
                   jax.ShapeDtypeStruct((2 * c, 2), jnp.float32)],
    )(sums, mask.reshape(mask.shape[1], 1),
      fc1_w, fc1_b.reshape(c, 1), fc2_w, fc2_b.reshape(c, 1),
      se1_w, se1_b.reshape(cr, 1), se2_w, se2_b.reshape(c, 1),
      conv1_w, conv1_b.reshape(k, 1), conv2_w, conv2_b.reshape(k, 1))

    out_lst, out_gui = pl.pallas_call(
        functools.partial(_transform_body, hb=hb),
        grid=(grid,),
        in_specs=[
            big_spec,
            big_spec,
            pl.BlockSpec((2 * c, 2 * c), lambda i: (0, 0)),
            pl.BlockSpec((2 * c, 2), lambda i: (0, 0)),
        ],
        out_specs=[big_spec, big_spec],
        out_shape=[jax.ShapeDtypeStruct((n, c, h, w), jnp.float32),
                   jax.ShapeDtypeStruct((n, c, h, w), jnp.float32)],
    )(lst, gui, a_mix, vec)

    m_out = vec[:c, 0].reshape(n, c)
    return out_lst, out_gui, m_out
